# baseline (speedup 1.0000x reference)
import functools

import jax
import jax.numpy as jnp
from jax import lax
from jax.experimental import pallas as pl
from jax.experimental.pallas import tpu as pltpu
from jax.experimental.pallas import tpu_sc as plsc

N = 10000
K = 16
MAX_ELEM = 100
ATOM_D = 128
DIR_IN = 10
DIR_D = 32
DIST_IN = 64
DIST_D = 64
SENDER_D = DIR_D + ATOM_D

NW = 32
N_PAD = 10240
ROWS_W = N_PAD // NW
GRP = 32
EDGES_GRP = GRP * K
N_GRPS = ROWS_W // GRP
EPS = 1e-5


def _silu(x):
    return x * (1.0 / (1.0 + jnp.exp(-x)))


def _node_body(an_ref, ndir_ref, mask_ref, nl_ref,
               src_tab_ref, tgt_tab_ref, wsd_ref, bsd_ref, wtd_ref, btd_ref,
               sender_ref, recv_ref, mnl_ref, inv_ref):
    bn = an_ref.shape[0]
    a = an_ref[...]
    oh = (lax.broadcasted_iota(jnp.int32, (bn, MAX_ELEM), 1) == a)
    oh = oh.astype(jnp.float32)
    sa = oh @ src_tab_ref[...]
    ta = oh @ tgt_tab_ref[...]
    nd = ndir_ref[...]
    sd = _silu(nd @ wsd_ref[...] + bsd_ref[...])
    td = _silu(nd @ wtd_ref[...] + btd_ref[...])
    m = mask_ref[...]
    cnt_raw = m.sum(axis=1, keepdims=True)
    cnt = cnt_raw + EPS
    inv = 1.0 / cnt
    scale = cnt_raw * inv
    pid = pl.program_id(0)
    rows = pid * bn + lax.broadcasted_iota(jnp.int32, (bn, 1), 0)
    validf = (rows < N).astype(jnp.float32)
    sender_ref[...] = jnp.concatenate([sd, sa], axis=1) * validf
    recv_ref[...] = jnp.concatenate([td, ta], axis=1) * scale
    mnl_ref[...] = jnp.where(m > 0.5, nl_ref[...], N)
    inv_ref[...] = jnp.broadcast_to(inv, (bn, K))


def _node_call(an2d, ndir, maskf, nl, src_tab, tgt_tab, wsd, bsd, wtd, btd):
    bn = 1024
    grid = (N_PAD // bn,)
    row_spec = lambda d: pl.BlockSpec((bn, d), lambda i: (i, 0))
    full = lambda shape: pl.BlockSpec(shape, lambda i: (0, 0))
    return pl.pallas_call(
        _node_body,
        grid=grid,
        in_specs=[
            row_spec(1), row_spec(DIR_IN), row_spec(K), row_spec(K),
            full((MAX_ELEM, ATOM_D)), full((MAX_ELEM, ATOM_D)),
            full((DIR_IN, DIR_D)), full((1, DIR_D)),
            full((DIR_IN, DIR_D)), full((1, DIR_D)),
        ],
        out_specs=[row_spec(SENDER_D), row_spec(SENDER_D), row_spec(K), row_spec(K)],
        out_shape=[
            jax.ShapeDtypeStruct((N_PAD, SENDER_D), jnp.float32),
            jax.ShapeDtypeStruct((N_PAD, SENDER_D), jnp.float32),
            jax.ShapeDtypeStruct((N_PAD, K), jnp.int32),
            jax.ShapeDtypeStruct((N_PAD, K), jnp.float32),
        ],
    )(an2d, ndir, maskf, nl, src_tab, tgt_tab, wsd, bsd, wtd, btd)


def _edge_body(e_ref, m_ref, wd_ref, bd_ref, out_ref):
    be = m_ref.shape[0]
    x = e_ref[...].reshape(be * K, DIST_IN)
    h = x @ wd_ref[...] + bd_ref[...]
    h = _silu(h).reshape(be, K, DIST_D)
    m3 = m_ref[...]
    cnt = m3.sum(axis=1, keepdims=True) + EPS
    out_ref[...] = (h * (m3 / cnt)).sum(axis=1)


def _edge_call(edist, mask3, wd, bd):
    be = 1000
    grid = (N // be,)
    return pl.pallas_call(
        _edge_body,
        grid=grid,
        in_specs=[
            pl.BlockSpec((be, K, DIST_IN), lambda i: (i, 0, 0)),
            pl.BlockSpec((be, K, 1), lambda i: (i, 0, 0)),
            pl.BlockSpec((DIST_IN, DIST_D), lambda i: (0, 0)),
            pl.BlockSpec((1, DIST_D), lambda i: (0, 0)),
        ],
        out_specs=pl.BlockSpec((be, DIST_D), lambda i: (i, 0)),
        out_shape=jax.ShapeDtypeStruct((N, DIST_D), jnp.float32),
    )(edist, mask3, wd, bd)


def _sc_body(sender_hbm, mnl_hbm, inv_hbm, out_hbm,
             idx_v, dst_v, out_v, inv_v, sem):
    c = lax.axis_index("c")
    s = lax.axis_index("s")
    wid = s * 2 + c
    base_n = wid * ROWS_W
    pltpu.sync_copy(inv_hbm.at[pl.ds(base_n, ROWS_W)], inv_v)

    @pl.loop(0, N_GRPS)
    def _group(g):
        row0 = wid * (ROWS_W * K // 128) + g * (EDGES_GRP // 128)
        pltpu.sync_copy(mnl_hbm.at[pl.ds(row0, EDGES_GRP // 128)], idx_v)
        cps = [
            pltpu.async_copy(
                sender_hbm.at[idx_v.at[j]],
                dst_v.at[pl.ds(j * 128, 128)],
                sem,
            )
            for j in range(EDGES_GRP // 128)
        ]
        for cp in cps:
            cp.wait()

        @pl.loop(0, GRP)
        def _node(n):
            e0 = n * K
            scale = inv_v[g * GRP + n, :]
            for cc in range(SENDER_D // 16):
                acc = dst_v[e0, pl.ds(cc * 16, 16)]
                for r in range(1, K):
                    acc = acc + dst_v[e0 + r, pl.ds(cc * 16, 16)]
                out_v[n, pl.ds(cc * 16, 16)] = acc * scale

        pltpu.sync_copy(out_v, out_hbm.at[pl.ds(base_n + g * GRP, GRP)])


def _sc_call(sender, mnl2d, inv_flat):
    mesh = plsc.VectorSubcoreMesh(core_axis_name="c", subcore_axis_name="s")
    f = pl.kernel(
        _sc_body,
        out_type=jax.ShapeDtypeStruct((N_PAD, SENDER_D), jnp.float32),
        mesh=mesh,
        scratch_types=[
            pltpu.VMEM((EDGES_GRP // 128, 128), jnp.int32),
            pltpu.VMEM((EDGES_GRP, SENDER_D), jnp.float32),
            pltpu.VMEM((GRP, SENDER_D), jnp.float32),
            pltpu.VMEM((ROWS_W, K), jnp.float32),
            pltpu.SemaphoreType.DMA,
        ],
        compiler_params=pltpu.CompilerParams(use_tc_tiling_on_sc=False),
    )
    return f(sender, mnl2d, inv_flat)


def kernel(atomic_numbers, node_direction_expansion, edge_distance_expansion,
           neighbor_list, neighbor_mask, src_atom_table, tgt_atom_table,
           W_src_dir, b_src_dir, W_tgt_dir, b_tgt_dir, W_dist, b_dist):
    pad = N_PAD - N
    an2d = jnp.pad(atomic_numbers.astype(jnp.int32), (0, pad)).reshape(N_PAD, 1)
    ndir = jnp.pad(node_direction_expansion, ((0, pad), (0, 0)))
    maskf = jnp.pad(neighbor_mask.astype(jnp.float32), ((0, pad), (0, 0)))
    nl = jnp.pad(neighbor_list.astype(jnp.int32), ((0, pad), (0, 0)))

    sender, recv, mnl, inv = _node_call(
        an2d, ndir, maskf, nl, src_atom_table, tgt_atom_table,
        W_src_dir, b_src_dir.reshape(1, DIR_D),
        W_tgt_dir, b_tgt_dir.reshape(1, DIR_D))

    out_de = _edge_call(edge_distance_expansion,
                        neighbor_mask.astype(jnp.float32).reshape(N, K, 1),
                        W_dist, b_dist.reshape(1, DIST_D))

    out_sc = _sc_call(sender, mnl.reshape(N_PAD * K // 128, 128), inv)

    return jnp.concatenate([out_de, out_sc[:N], recv[:N]], axis=1)


# speedup vs baseline: 10.4167x; 10.4167x over previous
import functools

import jax
import jax.numpy as jnp
from jax import lax
from jax.experimental import pallas as pl
from jax.experimental.pallas import tpu as pltpu
from jax.experimental.pallas import tpu_sc as plsc

N = 10000
K = 16
MAX_ELEM = 100
ELEM_PAD = 112
NULL_ELEM = 100
ATOM_D = 128
DIR_IN = 10
DIR_D = 32
DIST_IN = 64
DIST_D = 64
RECV_D = DIR_D + ATOM_D

NW = 32
N_PAD = 10240
ROWS_W = N_PAD // NW
GRP = 32
SD_Q = 4
SD_QW = DIR_D // SD_Q
EPS = 1e-5


def _silu(x):
    return x * (1.0 / (1.0 + jnp.exp(-x)))


def _node_body(an_ref, ndir_ref, mask_ref, nl_ref,
               tgt_tab_ref, wsd_ref, bsd_ref, wtd_ref, btd_ref,
               sd0_ref, sd1_ref, sd2_ref, sd3_ref,
               recv_ref, mnl_ref, inv_ref):
    bn = an_ref.shape[0]
    a = an_ref[...]
    oh = (lax.broadcasted_iota(jnp.int32, (bn, MAX_ELEM), 1) == a)
    oh = oh.astype(jnp.float32)
    ta = oh @ tgt_tab_ref[...]
    nd = ndir_ref[...]
    sd = _silu(nd @ wsd_ref[...] + bsd_ref[...])
    td = _silu(nd @ wtd_ref[...] + btd_ref[...])
    m = mask_ref[...]
    cnt_raw = m.sum(axis=1, keepdims=True)
    cnt = cnt_raw + EPS
    inv = 1.0 / cnt
    scale = cnt_raw * inv
    pid = pl.program_id(0)
    rows = pid * bn + lax.broadcasted_iota(jnp.int32, (bn, 1), 0)
    validf = (rows < N).astype(jnp.float32)
    sdv = sd * validf
    sd0_ref[...] = sdv[:, 0:8]
    sd1_ref[...] = sdv[:, 8:16]
    sd2_ref[...] = sdv[:, 16:24]
    sd3_ref[...] = sdv[:, 24:32]
    recv_ref[...] = jnp.concatenate([td, ta], axis=1) * scale
    mnl_ref[...] = jnp.where(m > 0.5, nl_ref[...], N)
    inv_ref[...] = jnp.broadcast_to(inv, (bn, K))


def _node_call(an2d, ndir, maskf, nl, tgt_tab, wsd, bsd, wtd, btd):
    bn = 1024
    grid = (N_PAD // bn,)
    row_spec = lambda d: pl.BlockSpec((bn, d), lambda i: (i, 0))
    full = lambda shape: pl.BlockSpec(shape, lambda i: (0, 0))
    return pl.pallas_call(
        _node_body,
        grid=grid,
        in_specs=[
            row_spec(1), row_spec(DIR_IN), row_spec(K), row_spec(K),
            full((MAX_ELEM, ATOM_D)),
            full((DIR_IN, DIR_D)), full((1, DIR_D)),
            full((DIR_IN, DIR_D)), full((1, DIR_D)),
        ],
        out_specs=[row_spec(SD_QW)] * SD_Q + [
            row_spec(RECV_D), row_spec(K), row_spec(K)],
        out_shape=[jax.ShapeDtypeStruct((N_PAD, SD_QW), jnp.float32)] * SD_Q + [
            jax.ShapeDtypeStruct((N_PAD, RECV_D), jnp.float32),
            jax.ShapeDtypeStruct((N_PAD, K), jnp.int32),
            jax.ShapeDtypeStruct((N_PAD, K), jnp.float32),
        ],
    )(an2d, ndir, maskf, nl, tgt_tab, wsd, bsd, wtd, btd)


def _edge_body(e_ref, m_ref, wd_ref, bd_ref, out_ref):
    be = m_ref.shape[0]
    x = e_ref[...].reshape(be * K, DIST_IN)
    h = x @ wd_ref[...] + bd_ref[...]
    h = _silu(h).reshape(be, K, DIST_D)
    m3 = m_ref[...]
    cnt = m3.sum(axis=1, keepdims=True) + EPS
    out_ref[...] = (h * (m3 / cnt)).sum(axis=1)


def _edge_call(edist, mask3, wd, bd):
    be = 1000
    grid = (N // be,)
    return pl.pallas_call(
        _edge_body,
        grid=grid,
        in_specs=[
            pl.BlockSpec((be, K, DIST_IN), lambda i: (i, 0, 0)),
            pl.BlockSpec((be, K, 1), lambda i: (i, 0, 0)),
            pl.BlockSpec((DIST_IN, DIST_D), lambda i: (0, 0)),
            pl.BlockSpec((1, DIST_D), lambda i: (0, 0)),
        ],
        out_specs=pl.BlockSpec((be, DIST_D), lambda i: (i, 0)),
        out_shape=jax.ShapeDtypeStruct((N, DIST_D), jnp.float32),
    )(edist, mask3, wd, bd)


def _hist_body(c_ref, tab_ref, inv_ref, out_ref):
    out_ref[...] = (c_ref[...] @ tab_ref[...]) * inv_ref[:, 0:1]


def _hist_call(counts, tab_pad, inv16):
    bn = 1024
    grid = (N_PAD // bn,)
    return pl.pallas_call(
        _hist_body,
        grid=grid,
        in_specs=[
            pl.BlockSpec((bn, ELEM_PAD), lambda i: (i, 0)),
            pl.BlockSpec((ELEM_PAD, ATOM_D), lambda i: (0, 0)),
            pl.BlockSpec((bn, K), lambda i: (i, 0)),
        ],
        out_specs=pl.BlockSpec((bn, ATOM_D), lambda i: (i, 0)),
        out_shape=jax.ShapeDtypeStruct((N_PAD, ATOM_D), jnp.float32),
    )(counts, tab_pad, inv16)


def _sc_body(an_hbm, mnl_hbm, inv_hbm, sd0_hbm, sd1_hbm, sd2_hbm, sd3_hbm,
             counts_hbm, outsd_hbm,
             an_v, mnl_v, inv_v, sdp_v, outsd_v, cnt_v):
    c = lax.axis_index("c")
    s = lax.axis_index("s")
    wid = s * 2 + c
    base = wid * ROWS_W
    pltpu.sync_copy(an_hbm, an_v)
    pltpu.sync_copy(mnl_hbm.at[pl.ds(base, ROWS_W)], mnl_v)
    pltpu.sync_copy(inv_hbm.at[pl.ds(base, ROWS_W)], inv_v)

    ones = jnp.ones((16,), jnp.float32)
    zeros = jnp.zeros((16,), jnp.float32)

    @pl.loop(0, ROWS_W // GRP)
    def _grp(g):
        @pl.loop(0, GRP)
        def _node(n):
            idx16 = mnl_v[g * GRP + n, :]
            eids = plsc.load_gather(an_v, [idx16])
            for cc in range(ELEM_PAD // 16):
                cnt_v[n, pl.ds(cc * 16, 16)] = zeros
            plsc.addupdate_scatter(
                cnt_v, [jnp.full((16,), n, jnp.int32), eids], ones)
        pltpu.sync_copy(cnt_v, counts_hbm.at[pl.ds(base + g * GRP, GRP)])

    lane = lax.broadcasted_iota(jnp.int32, (16,), 0)
    for p, sdp_hbm in enumerate((sd0_hbm, sd1_hbm, sd2_hbm, sd3_hbm)):
        pltpu.sync_copy(sdp_hbm, sdp_v)
        half = (p // 2) * 16
        lo = (p % 2) * SD_QW

        @pl.loop(0, ROWS_W)
        def _node_sd(n):
            idx16 = mnl_v[n, :]
            inv_s = inv_v[n, :][0]
            acc = zeros
            for f in range(SD_QW):
                v = plsc.load_gather(
                    sdp_v, [idx16, jnp.full((16,), f, jnp.int32)])
                oh = (lane == lo + f).astype(jnp.float32)
                acc = acc + oh * (jnp.sum(v) * inv_s)
            if p % 2 == 0:
                outsd_v[n, pl.ds(half, 16)] = acc
            else:
                outsd_v[n, pl.ds(half, 16)] = outsd_v[n, pl.ds(half, 16)] + acc

    pltpu.sync_copy(outsd_v, outsd_hbm.at[pl.ds(base, ROWS_W)])


def _sc_call(an_p, mnl, inv16, sd_quarters):
    mesh = plsc.VectorSubcoreMesh(core_axis_name="c", subcore_axis_name="s")
    f = pl.kernel(
        _sc_body,
        out_type=[
            jax.ShapeDtypeStruct((N_PAD, ELEM_PAD), jnp.float32),
            jax.ShapeDtypeStruct((N_PAD, DIR_D), jnp.float32),
        ],
        mesh=mesh,
        scratch_types=[
            pltpu.VMEM((N_PAD,), jnp.int32),
            pltpu.VMEM((ROWS_W, K), jnp.int32),
            pltpu.VMEM((ROWS_W, K), jnp.float32),
            pltpu.VMEM((N_PAD, SD_QW), jnp.float32),
            pltpu.VMEM((ROWS_W, DIR_D), jnp.float32),
            pltpu.VMEM((GRP, ELEM_PAD), jnp.float32),
        ],
        compiler_params=pltpu.CompilerParams(
            use_tc_tiling_on_sc=False, needs_layout_passes=False),
    )
    return f(an_p, mnl, inv16, *sd_quarters)


def kernel(atomic_numbers, node_direction_expansion, edge_distance_expansion,
           neighbor_list, neighbor_mask, src_atom_table, tgt_atom_table,
           W_src_dir, b_src_dir, W_tgt_dir, b_tgt_dir, W_dist, b_dist):
    pad = N_PAD - N
    an = atomic_numbers.astype(jnp.int32)
    an2d = jnp.pad(an, (0, pad)).reshape(N_PAD, 1)
    an_p = jnp.pad(an, (0, pad), constant_values=NULL_ELEM)
    ndir = jnp.pad(node_direction_expansion, ((0, pad), (0, 0)))
    maskf = jnp.pad(neighbor_mask.astype(jnp.float32), ((0, pad), (0, 0)))
    nl = jnp.pad(neighbor_list.astype(jnp.int32), ((0, pad), (0, 0)))

    (sd0, sd1, sd2, sd3, recv, mnl, inv16) = _node_call(
        an2d, ndir, maskf, nl, tgt_atom_table,
        W_src_dir, b_src_dir.reshape(1, DIR_D),
        W_tgt_dir, b_tgt_dir.reshape(1, DIR_D))

    out_de = _edge_call(edge_distance_expansion,
                        neighbor_mask.astype(jnp.float32).reshape(N, K, 1),
                        W_dist, b_dist.reshape(1, DIST_D))

    counts, out_sd = _sc_call(an_p, mnl, inv16, (sd0, sd1, sd2, sd3))

    tab_pad = jnp.pad(src_atom_table, ((0, ELEM_PAD - MAX_ELEM), (0, 0)))
    out_sa = _hist_call(counts, tab_pad, inv16)

    return jnp.concatenate(
        [out_de, out_sd[:N], out_sa[:N], recv[:N]], axis=1)


# speedup vs baseline: 12.1226x; 1.1638x over previous
import functools

import jax
import jax.numpy as jnp
from jax import lax
from jax.experimental import pallas as pl
from jax.experimental.pallas import tpu as pltpu
from jax.experimental.pallas import tpu_sc as plsc

N = 10000
K = 16
MAX_ELEM = 100
ELEM_PAD = 112
NULL_ELEM = 100
ATOM_D = 128
DIR_IN = 10
DIR_PAD = 10
DIR_D = 32
DIST_IN = 64
DIST_D = 64
RECV_D = DIR_D + ATOM_D
GND_D = K * DIR_PAD

NW = 32
N_PAD = 10240
ROWS_W = N_PAD // NW
CGRP = 32
GGRP = 16
EPS = 1e-5


def _silu(x):
    return x * (1.0 / (1.0 + jnp.exp(-x)))


def _node_body(an_ref, ndir_ref, mask_ref, nl_ref,
               tgt_tab_ref, wtd_ref, btd_ref,
               recv_ref, mnl_ref, inv_ref):
    bn = an_ref.shape[0]
    a = an_ref[...]
    oh = (lax.broadcasted_iota(jnp.int32, (bn, MAX_ELEM), 1) == a)
    oh = oh.astype(jnp.float32)
    ta = oh @ tgt_tab_ref[...]
    nd = ndir_ref[...]
    td = _silu(nd @ wtd_ref[...] + btd_ref[...])
    m = mask_ref[...]
    cnt_raw = m.sum(axis=1, keepdims=True)
    cnt = cnt_raw + EPS
    inv = 1.0 / cnt
    scale = cnt_raw * inv
    pid = pl.program_id(0)
    rows = pid * bn + lax.broadcasted_iota(jnp.int32, (bn, 1), 0)
    recv_ref[...] = jnp.concatenate([td, ta], axis=1) * scale
    mnl_ref[...] = jnp.where((m > 0.5) & (rows < N), nl_ref[...], N)
    inv_ref[...] = jnp.broadcast_to(inv, (bn, K))


def _node_call(an2d, ndir, maskf, nl, tgt_tab, wtd, btd):
    bn = 1024
    grid = (N_PAD // bn,)
    row_spec = lambda d: pl.BlockSpec((bn, d), lambda i: (i, 0))
    full = lambda shape: pl.BlockSpec(shape, lambda i: (0, 0))
    return pl.pallas_call(
        _node_body,
        grid=grid,
        in_specs=[
            row_spec(1), row_spec(DIR_IN), row_spec(K), row_spec(K),
            full((MAX_ELEM, ATOM_D)),
            full((DIR_IN, DIR_D)), full((1, DIR_D)),
        ],
        out_specs=[row_spec(RECV_D), row_spec(K), row_spec(K)],
        out_shape=[
            jax.ShapeDtypeStruct((N_PAD, RECV_D), jnp.float32),
            jax.ShapeDtypeStruct((N_PAD, K), jnp.int32),
            jax.ShapeDtypeStruct((N_PAD, K), jnp.float32),
        ],
    )(an2d, ndir, maskf, nl, tgt_tab, wtd, btd)


def _edge_body(e_ref, m_ref, wd_ref, bd_ref, out_ref):
    be = m_ref.shape[0]
    x = e_ref[...]
    wd = wd_ref[...].astype(jnp.bfloat16)
    h = jax.lax.dot(x.astype(jnp.bfloat16), wd,
                    preferred_element_type=jnp.float32) + bd_ref[...]
    h = _silu(h).reshape(be, K, DIST_D)
    m3 = m_ref[...]
    cnt = m3.sum(axis=1, keepdims=True) + EPS
    out_ref[...] = (h * (m3 / cnt)).sum(axis=1)


def _edge_call(edist2d, mask3, wd, bd):
    be = 1000
    grid = (N // be,)
    return pl.pallas_call(
        _edge_body,
        grid=grid,
        in_specs=[
            pl.BlockSpec((be * K, DIST_IN), lambda i: (i, 0)),
            pl.BlockSpec((be, K, 1), lambda i: (i, 0, 0)),
            pl.BlockSpec((DIST_IN, DIST_D), lambda i: (0, 0)),
            pl.BlockSpec((1, DIST_D), lambda i: (0, 0)),
        ],
        out_specs=pl.BlockSpec((be, DIST_D), lambda i: (i, 0)),
        out_shape=jax.ShapeDtypeStruct((N, DIST_D), jnp.float32),
    )(edist2d, mask3, wd, bd)


_SC_PARAMS = pltpu.CompilerParams(
    use_tc_tiling_on_sc=False, needs_layout_passes=False)


def _sc_hist_body(an_hbm, mnl_hbm, counts_hbm, an_v, mnl_v, cnt_v):
    c = lax.axis_index("c")
    s = lax.axis_index("s")
    wid = s * 2 + c
    base = wid * ROWS_W
    pltpu.sync_copy(an_hbm, an_v)
    pltpu.sync_copy(mnl_hbm.at[pl.ds(base, ROWS_W)], mnl_v)
    ones = jnp.ones((16,), jnp.float32)
    zeros = jnp.zeros((16,), jnp.float32)

    @pl.loop(0, ROWS_W // CGRP)
    def _cgrp(g):
        @pl.loop(0, CGRP)
        def _node(n):
            idx16 = mnl_v[g * CGRP + n, :]
            eids = plsc.load_gather(an_v, [idx16])
            for cc in range(ELEM_PAD // 16):
                cnt_v[n, pl.ds(cc * 16, 16)] = zeros
            plsc.addupdate_scatter(
                cnt_v, [jnp.full((16,), n, jnp.int32), eids], ones)
        pltpu.sync_copy(cnt_v, counts_hbm.at[pl.ds(base + g * CGRP, CGRP)])


def _sc_gnd_body(mnl_hbm, ndir_hbm, gnd_hbm, mnl_v, ndir_v, gnd_v):
    c = lax.axis_index("c")
    s = lax.axis_index("s")
    wid = s * 2 + c
    base = wid * ROWS_W
    pltpu.sync_copy(mnl_hbm.at[pl.ds(base, ROWS_W)], mnl_v)
    pltpu.sync_copy(ndir_hbm, ndir_v)
    lane = lax.broadcasted_iota(jnp.int32, (16,), 0)

    @pl.loop(0, ROWS_W // GGRP)
    def _ggrp(g):
        @pl.loop(0, GGRP)
        def _node(n):
            idx16 = jnp.minimum(mnl_v[g * GGRP + n, :], N - 1)
            base10 = idx16 * DIR_IN
            rown = jnp.full((16,), n, jnp.int32)
            for f in range(DIR_IN):
                v = plsc.load_gather(ndir_v, [base10 + f])
                plsc.store_scatter(gnd_v, [rown, lane * DIR_PAD + f], v)
        pltpu.sync_copy(gnd_v, gnd_hbm.at[pl.ds(base + g * GGRP, GGRP)])


def _sc_call(an_p, mnl, ndir):
    mesh = plsc.VectorSubcoreMesh(core_axis_name="c", subcore_axis_name="s")
    counts = pl.kernel(
        _sc_hist_body,
        out_type=jax.ShapeDtypeStruct((N_PAD, ELEM_PAD), jnp.float32),
        mesh=mesh,
        scratch_types=[
            pltpu.VMEM((N_PAD,), jnp.int32),
            pltpu.VMEM((ROWS_W, K), jnp.int32),
            pltpu.VMEM((CGRP, ELEM_PAD), jnp.float32),
        ],
        compiler_params=_SC_PARAMS,
    )(an_p, mnl)
    gnd = pl.kernel(
        _sc_gnd_body,
        out_type=jax.ShapeDtypeStruct((N_PAD, GND_D), jnp.float32),
        mesh=mesh,
        scratch_types=[
            pltpu.VMEM((ROWS_W, K), jnp.int32),
            pltpu.VMEM((N * DIR_IN,), jnp.float32),
            pltpu.VMEM((GGRP, GND_D), jnp.float32),
        ],
        compiler_params=_SC_PARAMS,
    )(mnl, ndir.reshape(N * DIR_IN))
    return counts, gnd


def _final_body(de_ref, cnt_ref, gnd_ref, mask_ref, inv_ref, recv_ref,
                tab_ref, wsd_ref, bsd_ref, out_ref):
    bn = mask_ref.shape[0]
    m = mask_ref[...]
    w = m / (m.sum(axis=1, keepdims=True) + EPS)
    g = gnd_ref[...]
    wsd = wsd_ref[...]
    bsd = bsd_ref[...]
    accsd = jnp.zeros((bn, DIR_D), jnp.float32)
    for k in range(K):
        gk = g[:, k * DIR_PAD:(k + 1) * DIR_PAD]
        accsd = accsd + _silu(gk @ wsd + bsd) * w[:, k:k + 1]
    sa = (cnt_ref[...] @ tab_ref[...]) * inv_ref[:, 0:1]
    out_ref[...] = jnp.concatenate(
        [de_ref[...], accsd, sa, recv_ref[...]], axis=1)


def _final_call(out_de, counts, gnd, maskf, inv16, recv, tab_pad, wsd_pad, bsd):
    bn = 1000
    grid = (N // bn,)
    row_spec = lambda d: pl.BlockSpec((bn, d), lambda i: (i, 0))
    full = lambda shape: pl.BlockSpec(shape, lambda i: (0, 0))
    return pl.pallas_call(
        _final_body,
        grid=grid,
        in_specs=[
            row_spec(DIST_D), row_spec(ELEM_PAD), row_spec(GND_D),
            row_spec(K), row_spec(K), row_spec(RECV_D),
            full((ELEM_PAD, ATOM_D)), full((DIR_PAD, DIR_D)),
            full((1, DIR_D)),
        ],
        out_specs=row_spec(DIST_D + DIR_D + ATOM_D + RECV_D),
        out_shape=jax.ShapeDtypeStruct((N, 384), jnp.float32),
    )(out_de, counts, gnd, maskf, inv16, recv, tab_pad, wsd_pad, bsd)


def kernel(atomic_numbers, node_direction_expansion, edge_distance_expansion,
           neighbor_list, neighbor_mask, src_atom_table, tgt_atom_table,
           W_src_dir, b_src_dir, W_tgt_dir, b_tgt_dir, W_dist, b_dist):
    an = atomic_numbers.astype(jnp.int32)
    an_p = jnp.pad(an, (0, N_PAD - N), constant_values=NULL_ELEM)
    maskf = neighbor_mask.astype(jnp.float32)
    nl = neighbor_list.astype(jnp.int32)

    recv, mnl, inv16 = _node_call(
        an.reshape(N, 1), node_direction_expansion, maskf, nl,
        tgt_atom_table, W_tgt_dir, b_tgt_dir.reshape(1, DIR_D))

    out_de = _edge_call(edge_distance_expansion.reshape(N * K, DIST_IN),
                        maskf.reshape(N, K, 1),
                        W_dist, b_dist.reshape(1, DIST_D))

    counts, gnd = _sc_call(an_p, mnl, node_direction_expansion)

    tab_pad = jnp.pad(src_atom_table, ((0, ELEM_PAD - MAX_ELEM), (0, 0)))
    wsd_pad = jnp.pad(W_src_dir, ((0, DIR_PAD - DIR_IN), (0, 0)))

    return _final_call(out_de, counts, gnd, maskf, inv16, recv,
                       tab_pad, wsd_pad, b_src_dir.reshape(1, DIR_D))


# speedup vs baseline: 12.3173x; 1.0161x over previous
import functools

import jax
import jax.numpy as jnp
from jax import lax
from jax.experimental import pallas as pl
from jax.experimental.pallas import tpu as pltpu
from jax.experimental.pallas import tpu_sc as plsc

N = 10000
K = 16
MAX_ELEM = 100
ELEM_PAD = 112
NULL_ELEM = 100
ATOM_D = 128
DIR_IN = 10
DIR_PAD = 10
DIR_D = 32
DIST_IN = 64
DIST_D = 64
RECV_D = DIR_D + ATOM_D
GND_D = K * DIR_PAD

NW = 32
N_PAD = 10240
ROWS_W = N_PAD // NW
CGRP = 32
GGRP = 64
NDIR_RB = 8
EPS = 1e-5


def _silu(x):
    return x * (1.0 / (1.0 + jnp.exp(-x)))


def _node_body(an_ref, ndir_ref, mask_ref, nl_ref,
               tgt_tab_ref, wtd_ref, btd_ref,
               recv_ref, mnl_ref, inv_ref):
    bn = an_ref.shape[0]
    a = an_ref[...]
    oh = (lax.broadcasted_iota(jnp.int32, (bn, MAX_ELEM), 1) == a)
    oh = oh.astype(jnp.float32)
    ta = oh @ tgt_tab_ref[...]
    nd = ndir_ref[...]
    td = _silu(nd @ wtd_ref[...] + btd_ref[...])
    m = mask_ref[...]
    cnt_raw = m.sum(axis=1, keepdims=True)
    cnt = cnt_raw + EPS
    inv = 1.0 / cnt
    scale = cnt_raw * inv
    pid = pl.program_id(0)
    rows = pid * bn + lax.broadcasted_iota(jnp.int32, (bn, 1), 0)
    recv_ref[...] = jnp.concatenate([td, ta], axis=1) * scale
    mnl_ref[...] = jnp.where((m > 0.5) & (rows < N), nl_ref[...], N)
    inv_ref[...] = jnp.broadcast_to(inv, (bn, K))


def _node_call(an2d, ndir, maskf, nl, tgt_tab, wtd, btd):
    bn = 1024
    grid = (N_PAD // bn,)
    row_spec = lambda d: pl.BlockSpec((bn, d), lambda i: (i, 0))
    full = lambda shape: pl.BlockSpec(shape, lambda i: (0, 0))
    return pl.pallas_call(
        _node_body,
        grid=grid,
        in_specs=[
            row_spec(1), row_spec(DIR_IN), row_spec(K), row_spec(K),
            full((MAX_ELEM, ATOM_D)),
            full((DIR_IN, DIR_D)), full((1, DIR_D)),
        ],
        out_specs=[row_spec(RECV_D), row_spec(K), row_spec(K)],
        out_shape=[
            jax.ShapeDtypeStruct((N_PAD, RECV_D), jnp.float32),
            jax.ShapeDtypeStruct((N_PAD, K), jnp.int32),
            jax.ShapeDtypeStruct((N_PAD, K), jnp.float32),
        ],
    )(an2d, ndir, maskf, nl, tgt_tab, wtd, btd)


def _edge_body(e_ref, m_ref, wd_ref, bd_ref, out_ref):
    be = m_ref.shape[0]
    x = e_ref[...]
    wd = wd_ref[...].astype(jnp.bfloat16)
    h = jax.lax.dot(x.astype(jnp.bfloat16), wd,
                    preferred_element_type=jnp.float32) + bd_ref[...]
    h = _silu(h).reshape(be, K, DIST_D)
    m3 = m_ref[...]
    cnt = m3.sum(axis=1, keepdims=True) + EPS
    out_ref[...] = (h * (m3 / cnt)).sum(axis=1)


def _edge_call(edist2d, mask3, wd, bd):
    be = 1000
    grid = (N // be,)
    return pl.pallas_call(
        _edge_body,
        grid=grid,
        in_specs=[
            pl.BlockSpec((be * K, DIST_IN), lambda i: (i, 0)),
            pl.BlockSpec((be, K, 1), lambda i: (i, 0, 0)),
            pl.BlockSpec((DIST_IN, DIST_D), lambda i: (0, 0)),
            pl.BlockSpec((1, DIST_D), lambda i: (0, 0)),
        ],
        out_specs=pl.BlockSpec((be, DIST_D), lambda i: (i, 0)),
        out_shape=jax.ShapeDtypeStruct((N, DIST_D), jnp.float32),
    )(edist2d, mask3, wd, bd)


_SC_PARAMS = pltpu.CompilerParams(
    use_tc_tiling_on_sc=False, needs_layout_passes=False)


def _sc_hist_body(an_hbm, mnl_hbm, counts_hbm, an_v, mnl_v, cnt_v):
    c = lax.axis_index("c")
    s = lax.axis_index("s")
    wid = s * 2 + c
    base = wid * ROWS_W
    pltpu.sync_copy(an_hbm, an_v)
    pltpu.sync_copy(mnl_hbm.at[pl.ds(base, ROWS_W)], mnl_v)
    ones = jnp.ones((16,), jnp.float32)
    zeros = jnp.zeros((16,), jnp.float32)

    @pl.loop(0, ROWS_W // CGRP)
    def _cgrp(g):
        @pl.loop(0, CGRP)
        def _node(n):
            idx16 = mnl_v[g * CGRP + n, :]
            eids = plsc.load_gather(an_v, [idx16])
            for cc in range(ELEM_PAD // 16):
                cnt_v[n, pl.ds(cc * 16, 16)] = zeros
            plsc.addupdate_scatter(
                cnt_v, [jnp.full((16,), n, jnp.int32), eids], ones)
        pltpu.sync_copy(cnt_v, counts_hbm.at[pl.ds(base + g * CGRP, CGRP)])


def _sc_gnd_body(mnl_hbm, ndir_hbm, gnd_hbm, mnl_v, ndir_v, gnd_v0, gnd_v1,
                 sem0, sem1):
    c = lax.axis_index("c")
    s = lax.axis_index("s")
    wid = s * 2 + c
    base = wid * ROWS_W
    pltpu.sync_copy(mnl_hbm.at[pl.ds(base, ROWS_W)], mnl_v)
    pltpu.sync_copy(ndir_hbm, ndir_v)
    lane = lax.broadcasted_iota(jnp.int32, (16,), 0)

    bufs = (gnd_v0, gnd_v1)
    sems = (sem0, sem1)
    n_grps = ROWS_W // GGRP
    cps = [None] * n_grps
    for g in range(n_grps):
        buf = bufs[g % 2]
        if g >= 2:
            cps[g - 2].wait()

        @pl.loop(0, GGRP)
        def _node(n):
            idx16 = jnp.minimum(mnl_v[g * GGRP + n, :], N - 1)
            rowi = jax.lax.shift_right_logical(idx16, 3)
            coli = (idx16 & 7) * DIR_IN
            rown = jnp.full((16,), n, jnp.int32)
            for f in range(DIR_IN):
                v = plsc.load_gather(ndir_v, [rowi, coli + f])
                plsc.store_scatter(buf, [rown, lane * DIR_PAD + f], v)

        cps[g] = pltpu.async_copy(
            buf, gnd_hbm.at[pl.ds(base + g * GGRP, GGRP)], sems[g % 2])
    for g in range(max(0, n_grps - 2), n_grps):
        cps[g].wait()


def _sc_call(an_p, mnl, ndir):
    mesh = plsc.VectorSubcoreMesh(core_axis_name="c", subcore_axis_name="s")
    counts = pl.kernel(
        _sc_hist_body,
        out_type=jax.ShapeDtypeStruct((N_PAD, ELEM_PAD), jnp.float32),
        mesh=mesh,
        scratch_types=[
            pltpu.VMEM((N_PAD,), jnp.int32),
            pltpu.VMEM((ROWS_W, K), jnp.int32),
            pltpu.VMEM((CGRP, ELEM_PAD), jnp.float32),
        ],
        compiler_params=_SC_PARAMS,
    )(an_p, mnl)
    gnd = pl.kernel(
        _sc_gnd_body,
        out_type=jax.ShapeDtypeStruct((N_PAD, GND_D), jnp.float32),
        mesh=mesh,
        scratch_types=[
            pltpu.VMEM((ROWS_W, K), jnp.int32),
            pltpu.VMEM((N // NDIR_RB, NDIR_RB * DIR_IN), jnp.float32),
            pltpu.VMEM((GGRP, GND_D), jnp.float32),
            pltpu.VMEM((GGRP, GND_D), jnp.float32),
            pltpu.SemaphoreType.DMA,
            pltpu.SemaphoreType.DMA,
        ],
        compiler_params=_SC_PARAMS,
    )(mnl, ndir.reshape(N // NDIR_RB, NDIR_RB * DIR_IN))
    return counts, gnd


def _final_body(de_ref, cnt_ref, gnd_ref, m_ref, inv_ref, recv_ref,
                tab_ref, wbd_ref, bsd_ref, out_ref):
    bn = m_ref.shape[0]
    m = m_ref[...]
    w = m / (m.sum(axis=1, keepdims=True) + EPS)
    g = gnd_ref[...].astype(jnp.bfloat16)
    h = jax.lax.dot(g, wbd_ref[...],
                    preferred_element_type=jnp.float32) + bsd_ref[...]
    h = _silu(h)
    accsd = jnp.zeros((bn, DIR_D), jnp.float32)
    for k in range(K):
        accsd = accsd + h[:, k * DIR_D:(k + 1) * DIR_D] * w[:, k:k + 1]
    sa = jax.lax.dot(cnt_ref[...].astype(jnp.bfloat16), tab_ref[...],
                     preferred_element_type=jnp.float32) * inv_ref[:, 0:1]
    out_ref[...] = jnp.concatenate(
        [de_ref[...], accsd, sa, recv_ref[...]], axis=1)


def _final_call(out_de, counts, gnd, maskf, inv16, recv, tab_pad, wbd, bsd16):
    bn = 1000
    grid = (N // bn,)
    row_spec = lambda d: pl.BlockSpec((bn, d), lambda i: (i, 0))
    full = lambda shape: pl.BlockSpec(shape, lambda i: (0, 0))
    return pl.pallas_call(
        _final_body,
        grid=grid,
        in_specs=[
            row_spec(DIST_D), row_spec(ELEM_PAD), row_spec(GND_D),
            row_spec(K), row_spec(K), row_spec(RECV_D),
            full((ELEM_PAD, ATOM_D)), full((GND_D, K * DIR_D)),
            full((1, K * DIR_D)),
        ],
        out_specs=row_spec(DIST_D + DIR_D + ATOM_D + RECV_D),
        out_shape=jax.ShapeDtypeStruct((N, 384), jnp.float32),
    )(out_de, counts, gnd, maskf, inv16, recv, tab_pad, wbd, bsd16)


def kernel(atomic_numbers, node_direction_expansion, edge_distance_expansion,
           neighbor_list, neighbor_mask, src_atom_table, tgt_atom_table,
           W_src_dir, b_src_dir, W_tgt_dir, b_tgt_dir, W_dist, b_dist):
    an = atomic_numbers.astype(jnp.int32)
    an_p = jnp.pad(an, (0, N_PAD - N), constant_values=NULL_ELEM)
    maskf = neighbor_mask.astype(jnp.float32)
    mask3 = maskf.reshape(N, K, 1)
    nl = neighbor_list.astype(jnp.int32)

    out_de = _edge_call(edge_distance_expansion.reshape(N * K, DIST_IN),
                        mask3, W_dist, b_dist.reshape(1, DIST_D))

    recv, mnl, inv16 = _node_call(
        an.reshape(N, 1), node_direction_expansion, maskf, nl,
        tgt_atom_table, W_tgt_dir, b_tgt_dir.reshape(1, DIR_D))

    counts, gnd = _sc_call(an_p, mnl, node_direction_expansion)

    tab_pad = jnp.pad(src_atom_table, ((0, ELEM_PAD - MAX_ELEM), (0, 0)))
    tab_bf = tab_pad.astype(jnp.bfloat16)
    wbd = jnp.kron(jnp.eye(K, dtype=jnp.float32), W_src_dir)
    wbd_bf = wbd.astype(jnp.bfloat16)
    bsd16 = jnp.tile(b_src_dir, K).reshape(1, K * DIR_D)

    return _final_call(out_de, counts, gnd, maskf, inv16, recv,
                       tab_bf, wbd_bf, bsd16)


# speedup vs baseline: 15.6252x; 1.2686x over previous
import functools

import jax
import jax.numpy as jnp
from jax import lax
from jax.experimental import pallas as pl
from jax.experimental.pallas import tpu as pltpu
from jax.experimental.pallas import tpu_sc as plsc

N = 10000
K = 16
MAX_ELEM = 100
ELEM_PAD = 112
NULL_ELEM = 100
ATOM_D = 128
DIR_IN = 10
DIR_PAD = 10
DIR_D = 32
DIST_IN = 64
DIST_D = 64
RECV_D = DIR_D + ATOM_D
GND_D = K * DIR_PAD

NW = 32
N_PAD = 10240
ROWS_W = N_PAD // NW
CGRP = 32
GGRP = 64
NDIR_RB = 8
EPS = 1e-5


def _silu(x):
    return x * (1.0 / (1.0 + jnp.exp(-x)))


def _node_body(an_ref, ndir_ref, mask_ref, nl_ref,
               tgt_tab_ref, wtd_ref, btd_ref,
               recv_ref, mnl_ref, inv_ref):
    bn = an_ref.shape[0]
    a = an_ref[...]
    oh = (lax.broadcasted_iota(jnp.int32, (bn, MAX_ELEM), 1) == a)
    oh = oh.astype(jnp.float32)
    ta = oh @ tgt_tab_ref[...]
    nd = ndir_ref[...]
    td = _silu(nd @ wtd_ref[...] + btd_ref[...])
    m = mask_ref[...]
    cnt_raw = m.sum(axis=1, keepdims=True)
    cnt = cnt_raw + EPS
    inv = 1.0 / cnt
    scale = cnt_raw * inv
    pid = pl.program_id(0)
    rows = pid * bn + lax.broadcasted_iota(jnp.int32, (bn, 1), 0)
    recv_ref[...] = jnp.concatenate([td, ta], axis=1) * scale
    mnl_ref[...] = jnp.where((m > 0.5) & (rows < N), nl_ref[...], N)
    inv_ref[...] = jnp.broadcast_to(inv, (bn, K))


def _node_call(an2d, ndir, maskf, nl, tgt_tab, wtd, btd):
    bn = 1024
    grid = (N_PAD // bn,)
    row_spec = lambda d: pl.BlockSpec((bn, d), lambda i: (i, 0))
    full = lambda shape: pl.BlockSpec(shape, lambda i: (0, 0))
    return pl.pallas_call(
        _node_body,
        grid=grid,
        in_specs=[
            row_spec(1), row_spec(DIR_IN), row_spec(K), row_spec(K),
            full((MAX_ELEM, ATOM_D)),
            full((DIR_IN, DIR_D)), full((1, DIR_D)),
        ],
        out_specs=[row_spec(RECV_D), row_spec(K), row_spec(K)],
        out_shape=[
            jax.ShapeDtypeStruct((N_PAD, RECV_D), jnp.float32),
            jax.ShapeDtypeStruct((N_PAD, K), jnp.int32),
            jax.ShapeDtypeStruct((N_PAD, K), jnp.float32),
        ],
    )(an2d, ndir, maskf, nl, tgt_tab, wtd, btd)


def _edge_body(e_ref, m_ref, wd_ref, bd_ref, out_ref):
    be = m_ref.shape[0]
    m = m_ref[...]
    w = m / (m.sum(axis=1, keepdims=True) + EPS)
    x = e_ref[...]
    wd = wd_ref[...].astype(jnp.bfloat16)
    acc = jnp.zeros((be, DIST_D), jnp.float32)
    for k in range(K):
        xk = x[:, k * DIST_IN:(k + 1) * DIST_IN]
        h = jax.lax.dot(xk.astype(jnp.bfloat16), wd,
                        preferred_element_type=jnp.float32) + bd_ref[...]
        acc = acc + _silu(h) * w[:, k:k + 1]
    out_ref[...] = acc


def _edge_call(edist2d, maskf, wd, bd):
    be = 1000
    grid = (N // be,)
    return pl.pallas_call(
        _edge_body,
        grid=grid,
        in_specs=[
            pl.BlockSpec((be, K * DIST_IN), lambda i: (i, 0)),
            pl.BlockSpec((be, K), lambda i: (i, 0)),
            pl.BlockSpec((DIST_IN, DIST_D), lambda i: (0, 0)),
            pl.BlockSpec((1, DIST_D), lambda i: (0, 0)),
        ],
        out_specs=pl.BlockSpec((be, DIST_D), lambda i: (i, 0)),
        out_shape=jax.ShapeDtypeStruct((N, DIST_D), jnp.float32),
    )(edist2d, maskf, wd, bd)


_SC_PARAMS = pltpu.CompilerParams(
    use_tc_tiling_on_sc=False, needs_layout_passes=False)


def _sc_hist_body(an_hbm, mnl_hbm, counts_hbm, an_v, mnl_v, cnt_v):
    c = lax.axis_index("c")
    s = lax.axis_index("s")
    wid = s * 2 + c
    base = wid * ROWS_W
    pltpu.sync_copy(an_hbm, an_v)
    pltpu.sync_copy(mnl_hbm.at[pl.ds(base, ROWS_W)], mnl_v)
    ones = jnp.ones((16,), jnp.float32)
    zeros = jnp.zeros((16,), jnp.float32)

    @pl.loop(0, ROWS_W // CGRP)
    def _cgrp(g):
        @pl.loop(0, CGRP)
        def _node(n):
            idx16 = mnl_v[g * CGRP + n, :]
            eids = plsc.load_gather(an_v, [idx16])
            for cc in range(ELEM_PAD // 16):
                cnt_v[n, pl.ds(cc * 16, 16)] = zeros
            plsc.addupdate_scatter(
                cnt_v, [jnp.full((16,), n, jnp.int32), eids], ones)
        pltpu.sync_copy(cnt_v, counts_hbm.at[pl.ds(base + g * CGRP, CGRP)])


def _sc_gnd_body(mnl_hbm, ndir_hbm, gnd_hbm, mnl_v, ndir_v, gnd_v0, gnd_v1,
                 sem0, sem1):
    c = lax.axis_index("c")
    s = lax.axis_index("s")
    wid = s * 2 + c
    base = wid * ROWS_W
    pltpu.sync_copy(mnl_hbm.at[pl.ds(base, ROWS_W)], mnl_v)
    pltpu.sync_copy(ndir_hbm, ndir_v)
    lane = lax.broadcasted_iota(jnp.int32, (16,), 0)

    bufs = (gnd_v0, gnd_v1)
    sems = (sem0, sem1)
    n_grps = ROWS_W // GGRP
    cps = [None] * n_grps
    for g in range(n_grps):
        buf = bufs[g % 2]
        if g >= 2:
            cps[g - 2].wait()

        @pl.loop(0, GGRP)
        def _node(n):
            idx16 = jnp.minimum(mnl_v[g * GGRP + n, :], N - 1)
            rowi = jax.lax.shift_right_logical(idx16, 3)
            coli = (idx16 & 7) * DIR_IN
            rown = jnp.full((16,), n, jnp.int32)
            for f in range(DIR_IN):
                v = plsc.load_gather(ndir_v, [rowi, coli + f])
                plsc.store_scatter(buf, [rown, lane * DIR_PAD + f], v)

        cps[g] = pltpu.async_copy(
            buf, gnd_hbm.at[pl.ds(base + g * GGRP, GGRP)], sems[g % 2])
    for g in range(max(0, n_grps - 2), n_grps):
        cps[g].wait()


def _sc_call(an_p, mnl, ndir):
    mesh = plsc.VectorSubcoreMesh(core_axis_name="c", subcore_axis_name="s")
    counts = pl.kernel(
        _sc_hist_body,
        out_type=jax.ShapeDtypeStruct((N_PAD, ELEM_PAD), jnp.float32),
        mesh=mesh,
        scratch_types=[
            pltpu.VMEM((N_PAD,), jnp.int32),
            pltpu.VMEM((ROWS_W, K), jnp.int32),
            pltpu.VMEM((CGRP, ELEM_PAD), jnp.float32),
        ],
        compiler_params=_SC_PARAMS,
    )(an_p, mnl)
    gnd = pl.kernel(
        _sc_gnd_body,
        out_type=jax.ShapeDtypeStruct((N_PAD, GND_D), jnp.float32),
        mesh=mesh,
        scratch_types=[
            pltpu.VMEM((ROWS_W, K), jnp.int32),
            pltpu.VMEM((N // NDIR_RB, NDIR_RB * DIR_IN), jnp.float32),
            pltpu.VMEM((GGRP, GND_D), jnp.float32),
            pltpu.VMEM((GGRP, GND_D), jnp.float32),
            pltpu.SemaphoreType.DMA,
            pltpu.SemaphoreType.DMA,
        ],
        compiler_params=_SC_PARAMS,
    )(mnl, ndir.reshape(N // NDIR_RB, NDIR_RB * DIR_IN))
    return counts, gnd


def _final_body(de_ref, cnt_ref, gnd_ref, m_ref, inv_ref, recv_ref,
                tab_ref, wbd_ref, bsd_ref, out_ref):
    bn = m_ref.shape[0]
    m = m_ref[...]
    w = m / (m.sum(axis=1, keepdims=True) + EPS)
    g = gnd_ref[...].astype(jnp.bfloat16)
    h = jax.lax.dot(g, wbd_ref[...],
                    preferred_element_type=jnp.float32) + bsd_ref[...]
    h = _silu(h)
    accsd = jnp.zeros((bn, DIR_D), jnp.float32)
    for k in range(K):
        accsd = accsd + h[:, k * DIR_D:(k + 1) * DIR_D] * w[:, k:k + 1]
    sa = jax.lax.dot(cnt_ref[...].astype(jnp.bfloat16), tab_ref[...],
                     preferred_element_type=jnp.float32) * inv_ref[:, 0:1]
    out_ref[...] = jnp.concatenate(
        [de_ref[...], accsd, sa, recv_ref[...]], axis=1)


def _final_call(out_de, counts, gnd, maskf, inv16, recv, tab_pad, wbd, bsd16):
    bn = 1000
    grid = (N // bn,)
    row_spec = lambda d: pl.BlockSpec((bn, d), lambda i: (i, 0))
    full = lambda shape: pl.BlockSpec(shape, lambda i: (0, 0))
    return pl.pallas_call(
        _final_body,
        grid=grid,
        in_specs=[
            row_spec(DIST_D), row_spec(ELEM_PAD), row_spec(GND_D),
            row_spec(K), row_spec(K), row_spec(RECV_D),
            full((ELEM_PAD, ATOM_D)), full((GND_D, K * DIR_D)),
            full((1, K * DIR_D)),
        ],
        out_specs=row_spec(DIST_D + DIR_D + ATOM_D + RECV_D),
        out_shape=jax.ShapeDtypeStruct((N, 384), jnp.float32),
    )(out_de, counts, gnd, maskf, inv16, recv, tab_pad, wbd, bsd16)


def kernel(atomic_numbers, node_direction_expansion, edge_distance_expansion,
           neighbor_list, neighbor_mask, src_atom_table, tgt_atom_table,
           W_src_dir, b_src_dir, W_tgt_dir, b_tgt_dir, W_dist, b_dist):
    an = atomic_numbers.astype(jnp.int32)
    an_p = jnp.pad(an, (0, N_PAD - N), constant_values=NULL_ELEM)
    maskf = neighbor_mask.astype(jnp.float32)
    mask3 = maskf.reshape(N, K, 1)
    nl = neighbor_list.astype(jnp.int32)

    out_de = _edge_call(edge_distance_expansion.reshape(N, K * DIST_IN),
                        maskf, W_dist, b_dist.reshape(1, DIST_D))

    recv, mnl, inv16 = _node_call(
        an.reshape(N, 1), node_direction_expansion, maskf, nl,
        tgt_atom_table, W_tgt_dir, b_tgt_dir.reshape(1, DIR_D))

    counts, gnd = _sc_call(an_p, mnl, node_direction_expansion)

    tab_pad = jnp.pad(src_atom_table, ((0, ELEM_PAD - MAX_ELEM), (0, 0)))
    tab_bf = tab_pad.astype(jnp.bfloat16)
    wbd = jnp.kron(jnp.eye(K, dtype=jnp.float32), W_src_dir)
    wbd_bf = wbd.astype(jnp.bfloat16)
    bsd16 = jnp.tile(b_src_dir, K).reshape(1, K * DIR_D)

    return _final_call(out_de, counts, gnd, maskf, inv16, recv,
                       tab_bf, wbd_bf, bsd16)


# speedup vs baseline: 15.7152x; 1.0058x over previous
import functools

import jax
import jax.numpy as jnp
from jax import lax
from jax.experimental import pallas as pl
from jax.experimental.pallas import tpu as pltpu
from jax.experimental.pallas import tpu_sc as plsc

N = 10000
K = 16
MAX_ELEM = 100
ELEM_PAD = 112
NULL_ELEM = 100
ATOM_D = 128
DIR_IN = 10
DIR_PAD = 10
DIR_D = 32
DIST_IN = 64
DIST_D = 64
RECV_D = DIR_D + ATOM_D
GND_D = K * DIR_PAD

NW = 32
N_PAD = 10240
ROWS_W = N_PAD // NW
CGRP = 32
GGRP = 64
NDIR_RB = 8
EPS = 1e-5


def _silu(x):
    return x * (1.0 / (1.0 + jnp.exp(-x)))


def _node_body(an_ref, ndir_ref, mask_ref, nl_ref,
               tgt_tab_ref, wtd_ref, btd_ref,
               recv_ref, mnl_ref, inv_ref):
    bn = an_ref.shape[0]
    a = an_ref[...]
    oh = (lax.broadcasted_iota(jnp.int32, (bn, MAX_ELEM), 1) == a)
    oh = oh.astype(jnp.float32)
    ta = oh @ tgt_tab_ref[...]
    nd = ndir_ref[...]
    td = _silu(nd @ wtd_ref[...] + btd_ref[...])
    m = mask_ref[...]
    cnt_raw = m.sum(axis=1, keepdims=True)
    cnt = cnt_raw + EPS
    inv = 1.0 / cnt
    scale = cnt_raw * inv
    pid = pl.program_id(0)
    rows = pid * bn + lax.broadcasted_iota(jnp.int32, (bn, 1), 0)
    recv_ref[...] = jnp.concatenate([td, ta], axis=1) * scale
    mnl_ref[...] = jnp.where((m > 0.5) & (rows < N), nl_ref[...], N)
    inv_ref[...] = jnp.broadcast_to(inv, (bn, K))


def _node_call(an2d, ndir, maskf, nl, tgt_tab, wtd, btd):
    bn = 2048
    grid = (N_PAD // bn,)
    row_spec = lambda d: pl.BlockSpec((bn, d), lambda i: (i, 0))
    full = lambda shape: pl.BlockSpec(shape, lambda i: (0, 0))
    return pl.pallas_call(
        _node_body,
        grid=grid,
        in_specs=[
            row_spec(1), row_spec(DIR_IN), row_spec(K), row_spec(K),
            full((MAX_ELEM, ATOM_D)),
            full((DIR_IN, DIR_D)), full((1, DIR_D)),
        ],
        out_specs=[row_spec(RECV_D), row_spec(K), row_spec(K)],
        out_shape=[
            jax.ShapeDtypeStruct((N_PAD, RECV_D), jnp.float32),
            jax.ShapeDtypeStruct((N_PAD, K), jnp.int32),
            jax.ShapeDtypeStruct((N_PAD, K), jnp.float32),
        ],
    )(an2d, ndir, maskf, nl, tgt_tab, wtd, btd)


def _edge_body(e_ref, m_ref, wd_ref, bd_ref, out_ref):
    be = m_ref.shape[0]
    m = m_ref[...]
    w = m / (m.sum(axis=1, keepdims=True) + EPS)
    x = e_ref[...]
    wd = wd_ref[...].astype(jnp.bfloat16)
    acc = jnp.zeros((be, DIST_D), jnp.float32)
    for k in range(K):
        xk = x[:, k * DIST_IN:(k + 1) * DIST_IN]
        h = jax.lax.dot(xk.astype(jnp.bfloat16), wd,
                        preferred_element_type=jnp.float32) + bd_ref[...]
        acc = acc + _silu(h) * w[:, k:k + 1]
    out_ref[...] = acc


def _edge_call(edist2d, maskf, wd, bd):
    be = 1000
    grid = (N // be,)
    return pl.pallas_call(
        _edge_body,
        grid=grid,
        in_specs=[
            pl.BlockSpec((be, K * DIST_IN), lambda i: (i, 0)),
            pl.BlockSpec((be, K), lambda i: (i, 0)),
            pl.BlockSpec((DIST_IN, DIST_D), lambda i: (0, 0)),
            pl.BlockSpec((1, DIST_D), lambda i: (0, 0)),
        ],
        out_specs=pl.BlockSpec((be, DIST_D), lambda i: (i, 0)),
        out_shape=jax.ShapeDtypeStruct((N, DIST_D), jnp.float32),
    )(edist2d, maskf, wd, bd)


_SC_PARAMS = pltpu.CompilerParams(
    use_tc_tiling_on_sc=False, needs_layout_passes=False)


def _sc_hist_body(an_hbm, mnl_hbm, counts_hbm, an_v, mnl_v, cnt_v):
    c = lax.axis_index("c")
    s = lax.axis_index("s")
    wid = s * 2 + c
    base = wid * ROWS_W
    pltpu.sync_copy(an_hbm, an_v)
    pltpu.sync_copy(mnl_hbm.at[pl.ds(base, ROWS_W)], mnl_v)
    ones = jnp.ones((16,), jnp.float32)
    zeros = jnp.zeros((16,), jnp.float32)

    @pl.loop(0, ROWS_W // CGRP)
    def _cgrp(g):
        @pl.loop(0, CGRP)
        def _node(n):
            idx16 = mnl_v[g * CGRP + n, :]
            eids = plsc.load_gather(an_v, [idx16])
            for cc in range(ELEM_PAD // 16):
                cnt_v[n, pl.ds(cc * 16, 16)] = zeros
            plsc.addupdate_scatter(
                cnt_v, [jnp.full((16,), n, jnp.int32), eids], ones)
        pltpu.sync_copy(cnt_v, counts_hbm.at[pl.ds(base + g * CGRP, CGRP)])


def _sc_gnd_body(mnl_hbm, ndir_hbm, gnd_hbm, mnl_v, ndir_v, gnd_v0, gnd_v1,
                 sem0, sem1):
    c = lax.axis_index("c")
    s = lax.axis_index("s")
    wid = s * 2 + c
    base = wid * ROWS_W
    pltpu.sync_copy(mnl_hbm.at[pl.ds(base, ROWS_W)], mnl_v)
    pltpu.sync_copy(ndir_hbm, ndir_v)
    lane = lax.broadcasted_iota(jnp.int32, (16,), 0)

    bufs = (gnd_v0, gnd_v1)
    sems = (sem0, sem1)
    n_grps = ROWS_W // GGRP
    cps = [None] * n_grps
    for g in range(n_grps):
        buf = bufs[g % 2]
        if g >= 2:
            cps[g - 2].wait()

        @pl.loop(0, GGRP)
        def _node(n):
            idx16 = jnp.minimum(mnl_v[g * GGRP + n, :], N - 1)
            rowi = jax.lax.shift_right_logical(idx16, 3)
            coli = (idx16 & 7) * DIR_IN
            rown = jnp.full((16,), n, jnp.int32)
            for f in range(DIR_IN):
                v = plsc.load_gather(ndir_v, [rowi, coli + f])
                plsc.store_scatter(buf, [rown, lane * DIR_PAD + f], v)

        cps[g] = pltpu.async_copy(
            buf, gnd_hbm.at[pl.ds(base + g * GGRP, GGRP)], sems[g % 2])
    for g in range(max(0, n_grps - 2), n_grps):
        cps[g].wait()


def _sc_call(an_p, mnl, ndir):
    mesh = plsc.VectorSubcoreMesh(core_axis_name="c", subcore_axis_name="s")
    counts = pl.kernel(
        _sc_hist_body,
        out_type=jax.ShapeDtypeStruct((N_PAD, ELEM_PAD), jnp.float32),
        mesh=mesh,
        scratch_types=[
            pltpu.VMEM((N_PAD,), jnp.int32),
            pltpu.VMEM((ROWS_W, K), jnp.int32),
            pltpu.VMEM((CGRP, ELEM_PAD), jnp.float32),
        ],
        compiler_params=_SC_PARAMS,
    )(an_p, mnl)
    gnd = pl.kernel(
        _sc_gnd_body,
        out_type=jax.ShapeDtypeStruct((N_PAD, GND_D), jnp.float32),
        mesh=mesh,
        scratch_types=[
            pltpu.VMEM((ROWS_W, K), jnp.int32),
            pltpu.VMEM((N // NDIR_RB, NDIR_RB * DIR_IN), jnp.float32),
            pltpu.VMEM((GGRP, GND_D), jnp.float32),
            pltpu.VMEM((GGRP, GND_D), jnp.float32),
            pltpu.SemaphoreType.DMA,
            pltpu.SemaphoreType.DMA,
        ],
        compiler_params=_SC_PARAMS,
    )(mnl, ndir.reshape(N // NDIR_RB, NDIR_RB * DIR_IN))
    return counts, gnd


def _final_body(de_ref, cnt_ref, gnd_ref, m_ref, inv_ref, recv_ref,
                tab_ref, wbd_ref, bsd_ref, out_ref):
    bn = m_ref.shape[0]
    m = m_ref[...]
    w = m / (m.sum(axis=1, keepdims=True) + EPS)
    g = gnd_ref[...].astype(jnp.bfloat16)
    h = jax.lax.dot(g, wbd_ref[...],
                    preferred_element_type=jnp.float32) + bsd_ref[...]
    h = _silu(h)
    accsd = jnp.zeros((bn, DIR_D), jnp.float32)
    for k in range(K):
        accsd = accsd + h[:, k * DIR_D:(k + 1) * DIR_D] * w[:, k:k + 1]
    sa = jax.lax.dot(cnt_ref[...].astype(jnp.bfloat16), tab_ref[...],
                     preferred_element_type=jnp.float32) * inv_ref[:, 0:1]
    out_ref[...] = jnp.concatenate(
        [de_ref[...], accsd, sa, recv_ref[...]], axis=1)


def _final_call(out_de, counts, gnd, maskf, inv16, recv, tab_pad, wbd, bsd16):
    bn = 2000
    grid = (N // bn,)
    row_spec = lambda d: pl.BlockSpec((bn, d), lambda i: (i, 0))
    full = lambda shape: pl.BlockSpec(shape, lambda i: (0, 0))
    return pl.pallas_call(
        _final_body,
        grid=grid,
        in_specs=[
            row_spec(DIST_D), row_spec(ELEM_PAD), row_spec(GND_D),
            row_spec(K), row_spec(K), row_spec(RECV_D),
            full((ELEM_PAD, ATOM_D)), full((GND_D, K * DIR_D)),
            full((1, K * DIR_D)),
        ],
        out_specs=row_spec(DIST_D + DIR_D + ATOM_D + RECV_D),
        out_shape=jax.ShapeDtypeStruct((N, 384), jnp.float32),
    )(out_de, counts, gnd, maskf, inv16, recv, tab_pad, wbd, bsd16)


def kernel(atomic_numbers, node_direction_expansion, edge_distance_expansion,
           neighbor_list, neighbor_mask, src_atom_table, tgt_atom_table,
           W_src_dir, b_src_dir, W_tgt_dir, b_tgt_dir, W_dist, b_dist):
    an = atomic_numbers.astype(jnp.int32)
    an_p = jnp.pad(an, (0, N_PAD - N), constant_values=NULL_ELEM)
    maskf = neighbor_mask.astype(jnp.float32)
    mask3 = maskf.reshape(N, K, 1)
    nl = neighbor_list.astype(jnp.int32)

    out_de = _edge_call(edge_distance_expansion.reshape(N, K * DIST_IN),
                        maskf, W_dist, b_dist.reshape(1, DIST_D))

    recv, mnl, inv16 = _node_call(
        an.reshape(N, 1), node_direction_expansion, maskf, nl,
        tgt_atom_table, W_tgt_dir, b_tgt_dir.reshape(1, DIR_D))

    counts, gnd = _sc_call(an_p, mnl, node_direction_expansion)

    tab_pad = jnp.pad(src_atom_table, ((0, ELEM_PAD - MAX_ELEM), (0, 0)))
    tab_bf = tab_pad.astype(jnp.bfloat16)
    wbd = jnp.kron(jnp.eye(K, dtype=jnp.float32), W_src_dir)
    wbd_bf = wbd.astype(jnp.bfloat16)
    bsd16 = jnp.tile(b_src_dir, K).reshape(1, K * DIR_D)

    return _final_call(out_de, counts, gnd, maskf, inv16, recv,
                       tab_bf, wbd_bf, bsd16)
